# baseline (speedup 1.0000x reference)
import functools

import jax
import jax.numpy as jnp
from jax import lax
from jax.experimental import pallas as pl
from jax.experimental.pallas import tpu as pltpu
from jax.experimental.pallas import tpu_sc as plsc

_THRESH = 0.35667494393873245
_C0 = -0.8472978603872036
_LN2 = 0.6931471805599453

_N = 16 * 512 * 512
_NMIN = int(_N * 0.15)
_NW = 32
_PER_W = _N // _NW
_CHUNK = 8192
_NCHUNK = _PER_W // _CHUNK
_L = 16
_FLUSH = 32
_NBLK = _CHUNK // (_L * _FLUSH)


def _vlog(p):
    bits = lax.bitcast_convert_type(p, jnp.int32)
    ex = ((bits >> 23) & 0xFF) - 127
    m = lax.bitcast_convert_type((bits & 0x7FFFFF) | 0x3F800000, jnp.float32)
    r = (m - 1.0) / (m + 1.0)
    r2 = r * r
    poly = 1.0 + r2 * (1.0 / 3.0 + r2 * (1.0 / 5.0 + r2 * (
        1.0 / 7.0 + r2 * (1.0 / 9.0 + r2 * (1.0 / 11.0 + r2 * (1.0 / 13.0))))))
    return ex.astype(jnp.float32) * _LN2 + (2.0 * r) * poly


def _accum_chunk(xbuf, ybuf, off0, vsum, vcnt, step_fn):
    def blk_body(blk, carry):
        vsum, vcnt = carry
        base = off0 + blk * (_L * _FLUSH)
        zeros = jnp.zeros((_L,), jnp.float32)
        ones = jnp.full((_L,), 1.0, jnp.float32)
        p = [ones, ones]
        sr = [zeros, zeros]
        sc = [zeros, zeros]
        for j in range(_FLUSH):
            k = j & 1
            o = base + j * _L
            x = xbuf[pl.ds(o, _L)]
            y = ybuf[pl.ds(o, _L)]
            t = x - 2.0 * (x * y)
            r = jnp.maximum(t, 0.0)
            e = jnp.exp(jnp.minimum(t, -t))
            s, rs = step_fn(t, r, e)
            p[k] = p[k] * (e * s + 1.0)
            sr[k] = sr[k] + rs
            sc[k] = sc[k] + s
        vsum = vsum + (sr[0] + sr[1]) + _vlog(p[0] * p[1])
        vcnt = vcnt + (sc[0] + sc[1])
        return (vsum, vcnt)

    return lax.fori_loop(0, _NBLK, blk_body, (vsum, vcnt))


def _fast_step(t, r, e):
    s = jnp.minimum(jnp.maximum((t - _C0) * 1.0995116e12, 0.0), 1.0)
    return s, r


def _thr_step(thv):
    def step(t, r, e):
        d = e - (jnp.exp(thv - r) - 1.0)
        s = jnp.minimum(jnp.maximum(d * 1.7e38, 0.0) * 1.7e38, 1.0)
        return s, r * s

    return step


def _stream_body(x_hbm, y_hbm, out_hbm, xbuf, ybuf, ostage, sem0, sem1,
                 step_fn):
    wid = lax.axis_index("s") * 2 + lax.axis_index("c")
    base = wid * _PER_W

    def start(g, slot, sem):
        pltpu.async_copy(x_hbm.at[pl.ds(base + g * _CHUNK, _CHUNK)],
                         xbuf.at[pl.ds(slot * _CHUNK, _CHUNK)], sem)
        pltpu.async_copy(y_hbm.at[pl.ds(base + g * _CHUNK, _CHUNK)],
                         ybuf.at[pl.ds(slot * _CHUNK, _CHUNK)], sem)

    def wait(slot, sem):
        pltpu.make_async_copy(x_hbm.at[pl.ds(base, _CHUNK)],
                              xbuf.at[pl.ds(slot * _CHUNK, _CHUNK)],
                              sem).wait()
        pltpu.make_async_copy(y_hbm.at[pl.ds(base, _CHUNK)],
                              ybuf.at[pl.ds(slot * _CHUNK, _CHUNK)],
                              sem).wait()

    start(0, 0, sem0)
    start(1, 1, sem1)

    def g_body(gp, carry):
        vsum, vcnt = carry
        for b, sem in ((0, sem0), (1, sem1)):
            g = gp * 2 + b
            wait(b, sem)
            vsum, vcnt = _accum_chunk(xbuf, ybuf, b * _CHUNK, vsum, vcnt,
                                      step_fn)

            @pl.when(g + 2 < _NCHUNK)
            def _():
                start(g + 2, b, sem)
        return (vsum, vcnt)

    vsum, vcnt = lax.fori_loop(
        0, _NCHUNK // 2, g_body,
        (jnp.zeros((_L,), jnp.float32), jnp.zeros((_L,), jnp.float32)))
    ostage[pl.ds(0, _L)] = vsum
    ostage[pl.ds(_L, _L)] = vcnt
    pltpu.sync_copy(ostage, out_hbm.at[wid])


_OUT = jax.ShapeDtypeStruct((_NW, 2 * _L), jnp.float32)


@functools.cache
def _fast_kernel():
    mesh = plsc.VectorSubcoreMesh(core_axis_name="c", subcore_axis_name="s")

    @functools.partial(
        pl.kernel, mesh=mesh, out_type=_OUT,
        scratch_types=[
            pltpu.VMEM((2 * _CHUNK,), jnp.float32),
            pltpu.VMEM((2 * _CHUNK,), jnp.float32),
            pltpu.VMEM((2 * _L,), jnp.float32),
            pltpu.SemaphoreType.DMA,
            pltpu.SemaphoreType.DMA,
        ])
    def k(x_hbm, y_hbm, out_hbm, xbuf, ybuf, ostage, sem0, sem1):
        _stream_body(x_hbm, y_hbm, out_hbm, xbuf, ybuf, ostage, sem0, sem1,
                     _fast_step)

    return k


@functools.cache
def _thr_kernel():
    mesh = plsc.VectorSubcoreMesh(core_axis_name="c", subcore_axis_name="s")

    @functools.partial(
        pl.kernel, mesh=mesh, out_type=_OUT,
        scratch_types=[
            pltpu.VMEM((2 * _CHUNK,), jnp.float32),
            pltpu.VMEM((2 * _CHUNK,), jnp.float32),
            pltpu.VMEM((_L,), jnp.float32),
            pltpu.VMEM((2 * _L,), jnp.float32),
            pltpu.SemaphoreType.DMA,
            pltpu.SemaphoreType.DMA,
        ])
    def k(x_hbm, y_hbm, t_hbm, out_hbm, xbuf, ybuf, tbuf, ostage,
          sem0, sem1):
        pltpu.sync_copy(t_hbm, tbuf)
        thv = tbuf[...]
        _stream_body(x_hbm, y_hbm, out_hbm, xbuf, ybuf, ostage, sem0, sem1,
                     _thr_step(thv))

    return k


def kernel(logits, labels):
    x = logits.reshape(-1)
    y = labels.reshape(-1)
    parts = _fast_kernel()(x, y)
    vsum = jnp.sum(parts[:, :_L])
    count = jnp.sum(parts[:, _L:])
    nminf = jnp.float32(_NMIN)

    def _masked(_):
        return vsum / count

    def _hard(_):
        def body(c):
            lo, hi = c
            mid = lo + (hi - lo) // 2
            tv = lax.bitcast_convert_type(mid, jnp.float32)
            p = _thr_kernel()(x, y, jnp.full((_L,), tv, jnp.float32))
            below = jnp.sum(p[:, _L:]) < nminf
            return (jnp.where(below, lo, mid + 1), jnp.where(below, mid, hi))

        lo, _ = lax.while_loop(lambda c: c[0] < c[1], body,
                               (jnp.int32(0), jnp.int32(0x7F7FFFFF)))
        v = lax.bitcast_convert_type(lo, jnp.float32)
        p = _thr_kernel()(x, y, jnp.full((_L,), v, jnp.float32))
        sum_gt = jnp.sum(p[:, :_L])
        cnt_gt = jnp.sum(p[:, _L:])
        return (sum_gt + (nminf - cnt_gt) * v) / nminf

    return lax.cond(count < nminf, _hard, _masked, 0)


# speedup vs baseline: 1.3784x; 1.3784x over previous
import functools

import jax
import jax.numpy as jnp
from jax import lax
from jax.experimental import pallas as pl
from jax.experimental.pallas import tpu as pltpu
from jax.experimental.pallas import tpu_sc as plsc

_THRESH = 0.35667494393873245
_C0 = -0.8472978603872036
_LN2 = 0.6931471805599453

_N = 16 * 512 * 512
_NMIN = int(_N * 0.15)
_NW = 32
_COLS = 512
_ROWS = _N // _COLS
_WROWS = _ROWS // _NW
_CROWS = 16
_NCHUNK = _WROWS // _CROWS
_L = 16
_RSTEPS = _COLS // _L


def _vlog(p):
    bits = lax.bitcast_convert_type(p, jnp.int32)
    ex = ((bits >> 23) & 0xFF) - 127
    m = lax.bitcast_convert_type((bits & 0x7FFFFF) | 0x3F800000, jnp.float32)
    r = (m - 1.0) / (m + 1.0)
    r2 = r * r
    poly = 1.0 + r2 * (1.0 / 3.0 + r2 * (1.0 / 5.0 + r2 * (
        1.0 / 7.0 + r2 * (1.0 / 9.0 + r2 * (1.0 / 11.0 + r2 * (1.0 / 13.0))))))
    return ex.astype(jnp.float32) * _LN2 + (2.0 * r) * poly


def _fast_step(t, r, e):
    s = jnp.minimum(jnp.maximum((t - _C0) * 1.0995116e12, 0.0), 1.0)
    return s, r


def _thr_step(thv):
    def step(t, r, e):
        d = e - (jnp.exp(thv - r) - 1.0)
        s = jnp.minimum(jnp.maximum(d * 1.7e38, 0.0) * 1.7e38, 1.0)
        return s, r * s

    return step


def _accum_chunk(xb, yb, vsum, vcnt, step_fn):
    def row_body(row, carry):
        vsum, vcnt = carry
        zeros = jnp.zeros((_L,), jnp.float32)
        ones = jnp.full((_L,), 1.0, jnp.float32)
        p = [ones, ones]
        sr = [zeros, zeros]
        sc = [zeros, zeros]
        for j in range(_RSTEPS):
            k = j & 1
            x = xb[row, pl.ds(j * _L, _L)]
            y = yb[row, pl.ds(j * _L, _L)]
            t = x - 2.0 * (x * y)
            r = jnp.maximum(t, 0.0)
            e = jnp.exp(jnp.minimum(t, -t))
            s, rs = step_fn(t, r, e)
            p[k] = p[k] * (e * s + 1.0)
            sr[k] = sr[k] + rs
            sc[k] = sc[k] + s
        vsum = vsum + (sr[0] + sr[1]) + _vlog(p[0] * p[1])
        vcnt = vcnt + (sc[0] + sc[1])
        return (vsum, vcnt)

    return lax.fori_loop(0, _CROWS, row_body, (vsum, vcnt))


def _stream_body(x_hbm, y_hbm, out_hbm, xb0, xb1, yb0, yb1, ostage,
                 sem0, sem1, step_fn):
    wid = lax.axis_index("s") * 2 + lax.axis_index("c")
    row0 = wid * _WROWS

    def start(g, xb, yb, sem):
        rows = pl.ds(row0 + g * _CROWS, _CROWS)
        pltpu.async_copy(x_hbm.at[rows, :], xb, sem)
        pltpu.async_copy(y_hbm.at[rows, :], yb, sem)

    def wait(xb, yb, sem):
        rows = pl.ds(row0, _CROWS)
        pltpu.make_async_copy(x_hbm.at[rows, :], xb, sem).wait()
        pltpu.make_async_copy(y_hbm.at[rows, :], yb, sem).wait()

    start(0, xb0, yb0, sem0)
    start(1, xb1, yb1, sem1)

    def g_body(gp, carry):
        vsum, vcnt = carry
        for b, xb, yb, sem in ((0, xb0, yb0, sem0), (1, xb1, yb1, sem1)):
            g = gp * 2 + b
            wait(xb, yb, sem)
            vsum, vcnt = _accum_chunk(xb, yb, vsum, vcnt, step_fn)

            @pl.when(g + 2 < _NCHUNK)
            def _():
                start(g + 2, xb, yb, sem)
        return (vsum, vcnt)

    vsum, vcnt = lax.fori_loop(
        0, _NCHUNK // 2, g_body,
        (jnp.zeros((_L,), jnp.float32), jnp.zeros((_L,), jnp.float32)))
    ostage[pl.ds(0, _L)] = vsum
    ostage[pl.ds(_L, _L)] = vcnt
    pltpu.sync_copy(ostage, out_hbm.at[wid])


_OUT = jax.ShapeDtypeStruct((_NW, 2 * _L), jnp.float32)


@functools.cache
def _fast_kernel():
    mesh = plsc.VectorSubcoreMesh(core_axis_name="c", subcore_axis_name="s")

    @functools.partial(
        pl.kernel, mesh=mesh, out_type=_OUT,
        scratch_types=[
            pltpu.VMEM((_CROWS, _COLS), jnp.float32),
            pltpu.VMEM((_CROWS, _COLS), jnp.float32),
            pltpu.VMEM((_CROWS, _COLS), jnp.float32),
            pltpu.VMEM((_CROWS, _COLS), jnp.float32),
            pltpu.VMEM((2 * _L,), jnp.float32),
            pltpu.SemaphoreType.DMA,
            pltpu.SemaphoreType.DMA,
        ])
    def k(x_hbm, y_hbm, out_hbm, xb0, xb1, yb0, yb1, ostage, sem0, sem1):
        _stream_body(x_hbm, y_hbm, out_hbm, xb0, xb1, yb0, yb1, ostage,
                     sem0, sem1, _fast_step)

    return k


@functools.cache
def _thr_kernel():
    mesh = plsc.VectorSubcoreMesh(core_axis_name="c", subcore_axis_name="s")

    @functools.partial(
        pl.kernel, mesh=mesh, out_type=_OUT,
        scratch_types=[
            pltpu.VMEM((_CROWS, _COLS), jnp.float32),
            pltpu.VMEM((_CROWS, _COLS), jnp.float32),
            pltpu.VMEM((_CROWS, _COLS), jnp.float32),
            pltpu.VMEM((_CROWS, _COLS), jnp.float32),
            pltpu.VMEM((_L,), jnp.float32),
            pltpu.VMEM((2 * _L,), jnp.float32),
            pltpu.SemaphoreType.DMA,
            pltpu.SemaphoreType.DMA,
        ])
    def k(x_hbm, y_hbm, t_hbm, out_hbm, xb0, xb1, yb0, yb1, tbuf, ostage,
          sem0, sem1):
        pltpu.sync_copy(t_hbm, tbuf)
        thv = tbuf[...]
        _stream_body(x_hbm, y_hbm, out_hbm, xb0, xb1, yb0, yb1, ostage,
                     sem0, sem1, _thr_step(thv))

    return k


def kernel(logits, labels):
    x = logits.reshape(_ROWS, _COLS)
    y = labels.reshape(_ROWS, _COLS)
    parts = _fast_kernel()(x, y)
    vsum = jnp.sum(parts[:, :_L])
    count = jnp.sum(parts[:, _L:])
    nminf = jnp.float32(_NMIN)

    def _masked(_):
        return vsum / count

    def _hard(_):
        def body(c):
            lo, hi = c
            mid = lo + (hi - lo) // 2
            tv = lax.bitcast_convert_type(mid, jnp.float32)
            p = _thr_kernel()(x, y, jnp.full((_L,), tv, jnp.float32))
            below = jnp.sum(p[:, _L:]) < nminf
            return (jnp.where(below, lo, mid + 1), jnp.where(below, mid, hi))

        lo, _ = lax.while_loop(lambda c: c[0] < c[1], body,
                               (jnp.int32(0), jnp.int32(0x7F7FFFFF)))
        v = lax.bitcast_convert_type(lo, jnp.float32)
        p = _thr_kernel()(x, y, jnp.full((_L,), v, jnp.float32))
        sum_gt = jnp.sum(p[:, :_L])
        cnt_gt = jnp.sum(p[:, _L:])
        return (sum_gt + (nminf - cnt_gt) * v) / nminf

    return lax.cond(count < nminf, _hard, _masked, 0)


# speedup vs baseline: 2.1269x; 1.5430x over previous
import functools

import jax
import jax.numpy as jnp
from jax import lax
from jax.experimental import pallas as pl
from jax.experimental.pallas import tpu as pltpu
from jax.experimental.pallas import tpu_sc as plsc

_THRESH = 0.35667494393873245
_C0 = -0.8472978603872036
_LN2 = 0.6931471805599453

_N = 16 * 512 * 512
_NMIN = int(_N * 0.15)
_NW = 32
_COLS = 512
_ROWS = _N // _COLS
_WROWS = _ROWS // _NW
_CROWS = 16
_NCHUNK = _WROWS // _CROWS
_L = 16
_RSTEPS = _COLS // _L


def _vlog(p):
    bits = lax.bitcast_convert_type(p, jnp.int32)
    ex = ((bits >> 23) & 0xFF) - 127
    m = lax.bitcast_convert_type((bits & 0x7FFFFF) | 0x3F800000, jnp.float32)
    r = (m - 1.0) / (m + 1.0)
    r2 = r * r
    poly = 1.0 + r2 * (1.0 / 3.0 + r2 * (1.0 / 5.0 + r2 * (
        1.0 / 7.0 + r2 * (1.0 / 9.0 + r2 * (1.0 / 11.0 + r2 * (1.0 / 13.0))))))
    return ex.astype(jnp.float32) * _LN2 + (2.0 * r) * poly


def _fast_step(t, r, e):
    s = jnp.minimum(jnp.maximum((t - _C0) * 1.0995116e12, 0.0), 1.0)
    return s, r


def _thr_step(thv):
    def step(t, r, e):
        d = e - (jnp.exp(thv - r) - 1.0)
        s = jnp.minimum(jnp.maximum(d * 1.7e38, 0.0) * 1.7e38, 1.0)
        return s, r * s

    return step


def _accum_chunk(xb, yb, vsum, vcnt, step_fn):
    def row_body(row, carry):
        vsum, vcnt = carry
        zeros = jnp.zeros((_L,), jnp.float32)
        ones = jnp.full((_L,), 1.0, jnp.float32)

        def half_body(jj, hc):
            p0, p1, sr0, sr1, sc0, sc1 = hc
            p = [p0, p1]
            sr = [sr0, sr1]
            sc = [sc0, sc1]
            base = jj * (_RSTEPS // 2 * _L)
            for j in range(_RSTEPS // 2):
                k = j & 1
                x = xb[row, pl.ds(base + j * _L, _L)]
                y = yb[row, pl.ds(base + j * _L, _L)]
                t = x - 2.0 * (x * y)
                r = jnp.maximum(t, 0.0)
                e = jnp.exp(jnp.minimum(t, -t))
                s, rs = step_fn(t, r, e)
                p[k] = p[k] * (e * s + 1.0)
                sr[k] = sr[k] + rs
                sc[k] = sc[k] + s
            return (p[0], p[1], sr[0], sr[1], sc[0], sc[1])

        p0, p1, sr0, sr1, sc0, sc1 = lax.fori_loop(
            0, 2, half_body, (ones, ones, zeros, zeros, zeros, zeros))
        vsum = vsum + (sr0 + sr1) + _vlog(p0 * p1)
        vcnt = vcnt + (sc0 + sc1)
        return (vsum, vcnt)

    return lax.fori_loop(0, _CROWS, row_body, (vsum, vcnt))


def _stream_body(x_hbm, y_hbm, out_hbm, xb0, xb1, yb0, yb1, ostage,
                 sem0, sem1, step_fn):
    wid = lax.axis_index("s") * 2 + lax.axis_index("c")
    row0 = wid * _WROWS

    def start(g, xb, yb, sem):
        rows = pl.ds(row0 + g * _CROWS, _CROWS)
        pltpu.async_copy(x_hbm.at[rows, :], xb, sem)
        pltpu.async_copy(y_hbm.at[rows, :], yb, sem)

    def wait(xb, yb, sem):
        rows = pl.ds(row0, _CROWS)
        pltpu.make_async_copy(x_hbm.at[rows, :], xb, sem).wait()
        pltpu.make_async_copy(y_hbm.at[rows, :], yb, sem).wait()

    start(0, xb0, yb0, sem0)
    start(1, xb1, yb1, sem1)

    def g_body(gp, carry):
        vsum, vcnt = carry
        for b, xb, yb, sem in ((0, xb0, yb0, sem0), (1, xb1, yb1, sem1)):
            g = gp * 2 + b
            wait(xb, yb, sem)
            vsum, vcnt = _accum_chunk(xb, yb, vsum, vcnt, step_fn)

            @pl.when(g + 2 < _NCHUNK)
            def _():
                start(g + 2, xb, yb, sem)
        return (vsum, vcnt)

    vsum, vcnt = lax.fori_loop(
        0, _NCHUNK // 2, g_body,
        (jnp.zeros((_L,), jnp.float32), jnp.zeros((_L,), jnp.float32)))
    ostage[pl.ds(0, _L)] = vsum
    ostage[pl.ds(_L, _L)] = vcnt
    pltpu.sync_copy(ostage, out_hbm.at[wid])


_OUT = jax.ShapeDtypeStruct((_NW, 2 * _L), jnp.float32)


@functools.cache
def _fast_kernel():
    mesh = plsc.VectorSubcoreMesh(core_axis_name="c", subcore_axis_name="s")

    @functools.partial(
        pl.kernel, mesh=mesh, out_type=_OUT,
        scratch_types=[
            pltpu.VMEM((_CROWS, _COLS), jnp.float32),
            pltpu.VMEM((_CROWS, _COLS), jnp.float32),
            pltpu.VMEM((_CROWS, _COLS), jnp.float32),
            pltpu.VMEM((_CROWS, _COLS), jnp.float32),
            pltpu.VMEM((2 * _L,), jnp.float32),
            pltpu.SemaphoreType.DMA,
            pltpu.SemaphoreType.DMA,
        ])
    def k(x_hbm, y_hbm, out_hbm, xb0, xb1, yb0, yb1, ostage, sem0, sem1):
        _stream_body(x_hbm, y_hbm, out_hbm, xb0, xb1, yb0, yb1, ostage,
                     sem0, sem1, _fast_step)

    return k


@functools.cache
def _thr_kernel():
    mesh = plsc.VectorSubcoreMesh(core_axis_name="c", subcore_axis_name="s")

    @functools.partial(
        pl.kernel, mesh=mesh, out_type=_OUT,
        scratch_types=[
            pltpu.VMEM((_CROWS, _COLS), jnp.float32),
            pltpu.VMEM((_CROWS, _COLS), jnp.float32),
            pltpu.VMEM((_CROWS, _COLS), jnp.float32),
            pltpu.VMEM((_CROWS, _COLS), jnp.float32),
            pltpu.VMEM((_L,), jnp.float32),
            pltpu.VMEM((2 * _L,), jnp.float32),
            pltpu.SemaphoreType.DMA,
            pltpu.SemaphoreType.DMA,
        ])
    def k(x_hbm, y_hbm, t_hbm, out_hbm, xb0, xb1, yb0, yb1, tbuf, ostage,
          sem0, sem1):
        pltpu.sync_copy(t_hbm, tbuf)
        thv = tbuf[...]
        _stream_body(x_hbm, y_hbm, out_hbm, xb0, xb1, yb0, yb1, ostage,
                     sem0, sem1, _thr_step(thv))

    return k


def kernel(logits, labels):
    x = logits.reshape(_ROWS, _COLS)
    y = labels.reshape(_ROWS, _COLS)
    parts = _fast_kernel()(x, y)
    vsum = jnp.sum(parts[:, :_L])
    count = jnp.sum(parts[:, _L:])
    nminf = jnp.float32(_NMIN)

    def _masked(_):
        return vsum / count

    def _hard(_):
        def body(c):
            lo, hi = c
            mid = lo + (hi - lo) // 2
            tv = lax.bitcast_convert_type(mid, jnp.float32)
            p = _thr_kernel()(x, y, jnp.full((_L,), tv, jnp.float32))
            below = jnp.sum(p[:, _L:]) < nminf
            return (jnp.where(below, lo, mid + 1), jnp.where(below, mid, hi))

        lo, _ = lax.while_loop(lambda c: c[0] < c[1], body,
                               (jnp.int32(0), jnp.int32(0x7F7FFFFF)))
        v = lax.bitcast_convert_type(lo, jnp.float32)
        p = _thr_kernel()(x, y, jnp.full((_L,), v, jnp.float32))
        sum_gt = jnp.sum(p[:, :_L])
        cnt_gt = jnp.sum(p[:, _L:])
        return (sum_gt + (nminf - cnt_gt) * v) / nminf

    return lax.cond(count < nminf, _hard, _masked, 0)


# speedup vs baseline: 3.2170x; 1.5125x over previous
import functools

import jax
import jax.numpy as jnp
from jax import lax
from jax.experimental import pallas as pl
from jax.experimental.pallas import tpu as pltpu
from jax.experimental.pallas import tpu_sc as plsc

_THRESH = 0.35667494393873245
_C0 = -0.8472978603872036
_LN2 = 0.6931471805599453

_N = 16 * 512 * 512
_NMIN = int(_N * 0.15)
_NW = 32
_COLS = 512
_ROWS = _N // _COLS
_CROWS = 16
_L = 16
_RSTEPS = _COLS // _L

_SC_ROWS = 3072
_TC_ROWS = _ROWS - _SC_ROWS
_TC_BROWS = 256
_TC_NBLK = _TC_ROWS // _TC_BROWS


def _vlog(p):
    bits = lax.bitcast_convert_type(p, jnp.int32)
    ex = ((bits >> 23) & 0xFF) - 127
    m = lax.bitcast_convert_type((bits & 0x7FFFFF) | 0x3F800000, jnp.float32)
    r = (m - 1.0) / (m + 1.0)
    r2 = r * r
    poly = 1.0 + r2 * (1.0 / 3.0 + r2 * (1.0 / 5.0 + r2 * (
        1.0 / 7.0 + r2 * (1.0 / 9.0 + r2 * (1.0 / 11.0 + r2 * (1.0 / 13.0))))))
    return ex.astype(jnp.float32) * _LN2 + (2.0 * r) * poly


def _fast_step(t, r, e):
    s = jnp.minimum(jnp.maximum((t - _C0) * 1.0995116e12, 0.0), 1.0)
    return s, r


def _thr_step(thv):
    def step(t, r, e):
        d = e - (jnp.exp(thv - r) - 1.0)
        s = jnp.minimum(jnp.maximum(d * 1.7e38, 0.0) * 1.7e38, 1.0)
        return s, r * s

    return step


def _accum_chunk(xb, yb, vsum, vcnt, step_fn):
    def row_body(row, carry):
        vsum, vcnt = carry
        zeros = jnp.zeros((_L,), jnp.float32)
        ones = jnp.full((_L,), 1.0, jnp.float32)

        def half_body(jj, hc):
            p0, p1, sr0, sr1, sc0, sc1 = hc
            p = [p0, p1]
            sr = [sr0, sr1]
            sc = [sc0, sc1]
            base = jj * (_RSTEPS // 2 * _L)
            for j in range(_RSTEPS // 2):
                k = j & 1
                x = xb[row, pl.ds(base + j * _L, _L)]
                y = yb[row, pl.ds(base + j * _L, _L)]
                t = x - 2.0 * (x * y)
                r = jnp.maximum(t, 0.0)
                e = jnp.exp(jnp.minimum(t, -t))
                s, rs = step_fn(t, r, e)
                p[k] = p[k] * (e * s + 1.0)
                sr[k] = sr[k] + rs
                sc[k] = sc[k] + s
            return (p[0], p[1], sr[0], sr[1], sc[0], sc[1])

        p0, p1, sr0, sr1, sc0, sc1 = lax.fori_loop(
            0, 2, half_body, (ones, ones, zeros, zeros, zeros, zeros))
        vsum = vsum + (sr0 + sr1) + _vlog(p0 * p1)
        vcnt = vcnt + (sc0 + sc1)
        return (vsum, vcnt)

    return lax.fori_loop(0, _CROWS, row_body, (vsum, vcnt))


def _stream_body(x_hbm, y_hbm, out_hbm, xb0, xb1, yb0, yb1, ostage,
                 sem0, sem1, step_fn, nchunk):
    wid = lax.axis_index("s") * 2 + lax.axis_index("c")
    row0 = wid * (nchunk * _CROWS)

    def start(g, xb, yb, sem):
        rows = pl.ds(row0 + g * _CROWS, _CROWS)
        pltpu.async_copy(x_hbm.at[rows, :], xb, sem)
        pltpu.async_copy(y_hbm.at[rows, :], yb, sem)

    def wait(xb, yb, sem):
        rows = pl.ds(row0, _CROWS)
        pltpu.make_async_copy(x_hbm.at[rows, :], xb, sem).wait()
        pltpu.make_async_copy(y_hbm.at[rows, :], yb, sem).wait()

    start(0, xb0, yb0, sem0)
    start(1, xb1, yb1, sem1)

    def g_body(gp, carry):
        vsum, vcnt = carry
        for b, xb, yb, sem in ((0, xb0, yb0, sem0), (1, xb1, yb1, sem1)):
            g = gp * 2 + b
            wait(xb, yb, sem)
            vsum, vcnt = _accum_chunk(xb, yb, vsum, vcnt, step_fn)

            @pl.when(g + 2 < nchunk)
            def _():
                start(g + 2, xb, yb, sem)
        return (vsum, vcnt)

    vsum, vcnt = lax.fori_loop(
        0, nchunk // 2, g_body,
        (jnp.zeros((_L,), jnp.float32), jnp.zeros((_L,), jnp.float32)))
    ostage[pl.ds(0, _L)] = vsum
    ostage[pl.ds(_L, _L)] = vcnt
    pltpu.sync_copy(ostage, out_hbm.at[wid])


_OUT = jax.ShapeDtypeStruct((_NW, 2 * _L), jnp.float32)


@functools.cache
def _fast_kernel():
    mesh = plsc.VectorSubcoreMesh(core_axis_name="c", subcore_axis_name="s")

    @functools.partial(
        pl.kernel, mesh=mesh, out_type=_OUT,
        scratch_types=[
            pltpu.VMEM((_CROWS, _COLS), jnp.float32),
            pltpu.VMEM((_CROWS, _COLS), jnp.float32),
            pltpu.VMEM((_CROWS, _COLS), jnp.float32),
            pltpu.VMEM((_CROWS, _COLS), jnp.float32),
            pltpu.VMEM((2 * _L,), jnp.float32),
            pltpu.SemaphoreType.DMA,
            pltpu.SemaphoreType.DMA,
        ])
    def k(x_hbm, y_hbm, out_hbm, xb0, xb1, yb0, yb1, ostage, sem0, sem1):
        _stream_body(x_hbm, y_hbm, out_hbm, xb0, xb1, yb0, yb1, ostage,
                     sem0, sem1, _fast_step, _SC_ROWS // _NW // _CROWS)

    return k


@functools.cache
def _thr_kernel():
    mesh = plsc.VectorSubcoreMesh(core_axis_name="c", subcore_axis_name="s")

    @functools.partial(
        pl.kernel, mesh=mesh, out_type=_OUT,
        scratch_types=[
            pltpu.VMEM((_CROWS, _COLS), jnp.float32),
            pltpu.VMEM((_CROWS, _COLS), jnp.float32),
            pltpu.VMEM((_CROWS, _COLS), jnp.float32),
            pltpu.VMEM((_CROWS, _COLS), jnp.float32),
            pltpu.VMEM((_L,), jnp.float32),
            pltpu.VMEM((2 * _L,), jnp.float32),
            pltpu.SemaphoreType.DMA,
            pltpu.SemaphoreType.DMA,
        ])
    def k(x_hbm, y_hbm, t_hbm, out_hbm, xb0, xb1, yb0, yb1, tbuf, ostage,
          sem0, sem1):
        pltpu.sync_copy(t_hbm, tbuf)
        thv = tbuf[...]
        _stream_body(x_hbm, y_hbm, out_hbm, xb0, xb1, yb0, yb1, ostage,
                     sem0, sem1, _thr_step(thv), _ROWS // _NW // _CROWS)

    return k


def _tc_body(xref, yref, oref, acc):
    g = pl.program_id(0)

    @pl.when(g == 0)
    def _():
        acc[0] = 0.0
        acc[1] = 0.0

    x = xref[...]
    y = yref[...]
    t = x - 2.0 * (x * y)
    r = jnp.maximum(t, 0.0)
    lg = jnp.log1p(jnp.exp(jnp.minimum(t, -t)))
    m = t > _C0
    acc[0] += jnp.sum(jnp.where(m, r + lg, 0.0))
    acc[1] += jnp.sum(m.astype(jnp.float32))

    @pl.when(g == _TC_NBLK - 1)
    def _():
        oref[0] = acc[0]
        oref[1] = acc[1]


def _tc_partials(x, y):
    return pl.pallas_call(
        _tc_body,
        grid=(_TC_NBLK,),
        in_specs=[
            pl.BlockSpec((_TC_BROWS, _COLS),
                         lambda g: (g + _SC_ROWS // _TC_BROWS, 0)),
            pl.BlockSpec((_TC_BROWS, _COLS),
                         lambda g: (g + _SC_ROWS // _TC_BROWS, 0)),
        ],
        out_specs=pl.BlockSpec(memory_space=pltpu.SMEM),
        out_shape=jax.ShapeDtypeStruct((2,), jnp.float32),
        scratch_shapes=[pltpu.SMEM((2,), jnp.float32)],
    )(x, y)


def kernel(logits, labels):
    x = logits.reshape(_ROWS, _COLS)
    y = labels.reshape(_ROWS, _COLS)
    parts = _fast_kernel()(x, y)
    tc = _tc_partials(x, y)
    vsum = jnp.sum(parts[:, :_L]) + tc[0]
    count = jnp.sum(parts[:, _L:]) + tc[1]
    nminf = jnp.float32(_NMIN)

    def _masked(_):
        return vsum / count

    def _hard(_):
        def body(c):
            lo, hi = c
            mid = lo + (hi - lo) // 2
            tv = lax.bitcast_convert_type(mid, jnp.float32)
            p = _thr_kernel()(x, y, jnp.full((_L,), tv, jnp.float32))
            below = jnp.sum(p[:, _L:]) < nminf
            return (jnp.where(below, lo, mid + 1), jnp.where(below, mid, hi))

        lo, _ = lax.while_loop(lambda c: c[0] < c[1], body,
                               (jnp.int32(0), jnp.int32(0x7F7FFFFF)))
        v = lax.bitcast_convert_type(lo, jnp.float32)
        p = _thr_kernel()(x, y, jnp.full((_L,), v, jnp.float32))
        sum_gt = jnp.sum(p[:, :_L])
        cnt_gt = jnp.sum(p[:, _L:])
        return (sum_gt + (nminf - cnt_gt) * v) / nminf

    return lax.cond(count < nminf, _hard, _masked, 0)
